# baseline (speedup 1.0000x reference)
import jax
import jax.numpy as jnp
from jax import lax
from jax.experimental import pallas as pl
from jax.experimental.pallas import tpu as pltpu
from jax.experimental.pallas import tpu_sc as plsc

NC = 2
NS = 16
NW = NC * NS
CH = 128
BB = 2048


def _sc_gather_body(pt_hbm, mt_hbm, x_hbm, out_hbm, x_v, idx_v, gbuf_v,
                    ebuf_v, gsem, ssem):
    wid = lax.axis_index("s") * NC + lax.axis_index("c")
    bpw = x_v.shape[0]
    nch = bpw // CH
    rb_base = wid * (bpw // 8)
    base = wid * bpw

    pltpu.sync_copy(x_hbm.at[pl.ds(base, bpw)], x_v)

    @plsc.parallel_loop(0, 6 * (bpw // 16), unroll=4)
    def extract(i):
        j = i // (bpw // 16)
        g = i % (bpw // 16)
        r16 = lax.iota(jnp.int32, 16) + g * 16
        ids = plsc.load_gather(x_v, [r16, jnp.full((16,), j, jnp.int32)])
        plsc.store_scatter(idx_v, [jnp.full((16,), j, jnp.int32),
                                   r16], ids.astype(jnp.int32))

    def zrow(i, _):
        rb, r = i // 8, i % 8
        for p in range(2):
            for q in range(4):
                ebuf_v[p, rb, 1, r, pl.ds(64 + 16 * q, 16)] = (
                    jnp.zeros((16,), jnp.float32))
        return 0
    lax.fori_loop(0, (CH // 8) * 8, zrow, 0)

    store = [None, None]
    for k in range(nch):
        copies = []
        for j in range(6):
            tab = pt_hbm if j < 2 else mt_hbm
            copies.append(
                pltpu.async_copy(tab.at[idx_v.at[j, pl.ds(k * CH, CH)]],
                                 gbuf_v.at[j], gsem))
        for c in copies:
            c.wait()
        if store[k % 2] is not None:
            store[k % 2].wait()

        ebuf = ebuf_v.at[k % 2]

        @plsc.parallel_loop(0, CH, unroll=4)
        def rearrange(rr):
            rbq, rq = rr // 8, rr % 8
            for j in range(6):
                for h in range(2):
                    lq = 32 * j + 16 * h
                    v = gbuf_v[j, rr, pl.ds(16 * h, 16)]
                    ebuf[rbq, lq // 128, rq, pl.ds(lq % 128, 16)] = v

        @plsc.parallel_loop(0, CH // 16, unroll=2)
        def numeric(g):
            r16 = lax.iota(jnp.int32, 16) + g * 16
            rbq16 = r16 // 8
            rq16 = r16 % 8
            for c in range(4):
                vals = plsc.load_gather(
                    x_v, [r16 + k * CH, jnp.full((16,), 6 + c, jnp.int32)])
                plsc.store_scatter(
                    ebuf, [rbq16, jnp.full((16,), 1, jnp.int32), rq16,
                           jnp.full((16,), 64 + c, jnp.int32)], vals)

        store[k % 2] = pltpu.async_copy(
            ebuf_v.at[k % 2],
            out_hbm.at[pl.ds(rb_base + k * (CH // 8), CH // 8)], ssem)
    for cp in store:
        if cp is not None:
            cp.wait()


def _mlp_body(act_ref, w1a_ref, w1b_ref, b1_ref, w2_ref, b2_ref, w3_ref,
              b3_ref, out_ref):
    nb = act_ref.shape[0] * 8
    lo = act_ref[:, 0].reshape(nb, 128)
    hi = act_ref[:, 1].reshape(nb, 128)
    h1 = jnp.dot(lo, w1a_ref[:, :], preferred_element_type=jnp.float32)
    h1 += jnp.dot(hi, w1b_ref[:, :], preferred_element_type=jnp.float32)
    h1 = jnp.maximum(h1 + b1_ref[:, :], 0.0)
    h2 = jnp.maximum(jnp.dot(h1, w2_ref[:, :],
                             preferred_element_type=jnp.float32)
                     + b2_ref[:, :], 0.0)
    out_ref[:, :] = (jnp.dot(h2, w3_ref[:, :],
                             preferred_element_type=jnp.float32)
                     + b3_ref[:, :])


def kernel(x, pokemon_table, move_table, W1, b1, W2, b2, W3, b3):
    B = x.shape[0]
    bpw = B // NW

    acts = pl.kernel(
        _sc_gather_body,
        out_type=jax.ShapeDtypeStruct((B // 8, 2, 8, 128), jnp.float32),
        scratch_types=[
            pltpu.VMEM((bpw, 10), jnp.float32),
            pltpu.VMEM((6, bpw), jnp.int32),
            pltpu.VMEM((6, CH, 32), jnp.float32),
            pltpu.VMEM((2, CH // 8, 2, 8, 128), jnp.float32),
            pltpu.SemaphoreType.DMA,
            pltpu.SemaphoreType.DMA,
        ],
        mesh=plsc.VectorSubcoreMesh(core_axis_name="c", subcore_axis_name="s"),
        compiler_params=pltpu.CompilerParams(
            use_tc_tiling_on_sc=False, needs_layout_passes=False),
    )(pokemon_table, move_table, x)

    W1a = W1[:128]
    W1b = jnp.concatenate(
        [W1[128:196], jnp.zeros((60, 128), W1.dtype)], axis=0)

    grid = B // BB
    full = lambda shape: pl.BlockSpec(shape, lambda i: (0,) * len(shape))
    return pl.pallas_call(
        _mlp_body,
        grid=(grid,),
        in_specs=[
            pl.BlockSpec((BB // 8, 2, 8, 128), lambda i: (i, 0, 0, 0)),
            full((128, 128)),
            full((128, 128)),
            full((1, 128)),
            full(W2.shape),
            full((1, 64)),
            full(W3.shape),
            full((1, 18)),
        ],
        out_specs=pl.BlockSpec((BB, 18), lambda i: (i, 0)),
        out_shape=jax.ShapeDtypeStruct((B, 18), jnp.float32),
    )(acts, W1a, W1b, b1.reshape(1, 128), W2, b2.reshape(1, 64), W3,
      b3.reshape(1, 18))


# speedup vs baseline: 1.2443x; 1.2443x over previous
import jax
import jax.numpy as jnp
from jax import lax
from jax.experimental import pallas as pl
from jax.experimental.pallas import tpu as pltpu
from jax.experimental.pallas import tpu_sc as plsc

NC = 2
NS = 16
NW = NC * NS
CH = 128
BB = 2048


def _sc_gather_body(pt_hbm, mt_hbm, idx_hbm, out_hbm, idx_v, gbuf_v, ebuf_v,
                    gsem, ssem):
    wid = lax.axis_index("s") * NC + lax.axis_index("c")
    nch = idx_v.shape[0] // 6
    bpw = nch * CH
    rb_base = wid * (bpw // 8)

    pltpu.sync_copy(idx_hbm.at[wid], idx_v)

    def zrow(i, _):
        rb, r = i // 8, i % 8
        for p in range(2):
            for q in range(4):
                ebuf_v[p, rb, 1, r, pl.ds(64 + 16 * q, 16)] = (
                    jnp.zeros((16,), jnp.float32))
        return 0
    lax.fori_loop(0, (CH // 8) * 8, zrow, 0)

    def fire(k):
        copies = []
        for j in range(6):
            tab = pt_hbm if j < 2 else mt_hbm
            copies.append(
                pltpu.async_copy(tab.at[idx_v.at[j * nch + k]],
                                 gbuf_v.at[k % 2, j], gsem))
        return copies

    store = [None, None]
    gcop = [None, None]
    gcop[0] = fire(0)
    for k in range(nch):
        if k + 1 < nch:
            gcop[(k + 1) % 2] = fire(k + 1)
        for c in gcop[k % 2]:
            c.wait()
        if store[k % 2] is not None:
            store[k % 2].wait()

        ebuf = ebuf_v.at[k % 2]
        gbuf = gbuf_v.at[k % 2]

        @plsc.parallel_loop(0, CH, unroll=4)
        def rearrange(rr):
            rbq, rq = rr // 8, rr % 8
            for j in range(6):
                for h in range(2):
                    lq = 32 * j + 16 * h
                    v = gbuf[j, rr, pl.ds(16 * h, 16)]
                    ebuf[rbq, lq // 128, rq, pl.ds(lq % 128, 16)] = v

        store[k % 2] = pltpu.async_copy(
            ebuf_v.at[k % 2],
            out_hbm.at[pl.ds(rb_base + k * (CH // 8), CH // 8)], ssem)
    for cp in store:
        if cp is not None:
            cp.wait()


def _mlp_body(act_ref, x_ref, w1a_ref, w1b_ref, w1n_ref, b1_ref, w2_ref,
              b2_ref, w3_ref, b3_ref, out_ref):
    nb = act_ref.shape[0] * 8
    lo = act_ref[:, 0].reshape(nb, 128)
    hi = act_ref[:, 1].reshape(nb, 128)
    h1 = jnp.dot(lo, w1a_ref[:, :], preferred_element_type=jnp.float32)
    h1 += jnp.dot(hi, w1b_ref[:, :], preferred_element_type=jnp.float32)
    h1 += jnp.dot(x_ref[:, 6:10], w1n_ref[:, :],
                  preferred_element_type=jnp.float32)
    h1 = jnp.maximum(h1 + b1_ref[:, :], 0.0)
    h2 = jnp.maximum(jnp.dot(h1, w2_ref[:, :],
                             preferred_element_type=jnp.float32)
                     + b2_ref[:, :], 0.0)
    out_ref[:, :] = (jnp.dot(h2, w3_ref[:, :],
                             preferred_element_type=jnp.float32)
                     + b3_ref[:, :])


def kernel(x, pokemon_table, move_table, W1, b1, W2, b2, W3, b3):
    B = x.shape[0]
    bpw = B // NW
    nch = bpw // CH
    ids = x[:, :6].astype(jnp.int32)
    idx = ids.T.reshape(6, NW, nch, CH).transpose(1, 0, 2, 3)
    idx = idx.reshape(NW, 6 * nch, CH)

    acts = pl.kernel(
        _sc_gather_body,
        out_type=jax.ShapeDtypeStruct((B // 8, 2, 8, 128), jnp.float32),
        scratch_types=[
            pltpu.VMEM((6 * nch, CH), jnp.int32),
            pltpu.VMEM((2, 6, CH, 32), jnp.float32),
            pltpu.VMEM((2, CH // 8, 2, 8, 128), jnp.float32),
            pltpu.SemaphoreType.DMA,
            pltpu.SemaphoreType.DMA,
        ],
        mesh=plsc.VectorSubcoreMesh(core_axis_name="c", subcore_axis_name="s"),
        compiler_params=pltpu.CompilerParams(
            use_tc_tiling_on_sc=False, needs_layout_passes=False),
    )(pokemon_table, move_table, idx)

    W1a = W1[:128]
    W1b = jnp.pad(W1[128:192], ((0, 64), (0, 0)))
    W1n = W1[192:196]

    grid = B // BB
    full = lambda shape: pl.BlockSpec(shape, lambda i: (0,) * len(shape))
    return pl.pallas_call(
        _mlp_body,
        grid=(grid,),
        in_specs=[
            pl.BlockSpec((BB // 8, 2, 8, 128), lambda i: (i, 0, 0, 0)),
            pl.BlockSpec((BB, 10), lambda i: (i, 0)),
            full((128, 128)),
            full((128, 128)),
            full((4, 128)),
            full((1, 128)),
            full(W2.shape),
            full((1, 64)),
            full(W3.shape),
            full((1, 18)),
        ],
        out_specs=pl.BlockSpec((BB, 18), lambda i: (i, 0)),
        out_shape=jax.ShapeDtypeStruct((B, 18), jnp.float32),
    )(acts, x, W1a, W1b, W1n, b1.reshape(1, 128), W2, b2.reshape(1, 64), W3,
      b3.reshape(1, 18))


# speedup vs baseline: 1.2773x; 1.0266x over previous
import jax
import jax.numpy as jnp
from jax import lax
from jax.experimental import pallas as pl
from jax.experimental.pallas import tpu as pltpu
from jax.experimental.pallas import tpu_sc as plsc

NC = 2
NS = 16
NW = NC * NS
CH = 128
BB = 4096


def _sc_gather_body(pt_hbm, mt_hbm, idx_hbm, out_hbm, idx_v, gbuf_v, ebuf_v,
                    gsem, ssem):
    wid = lax.axis_index("s") * NC + lax.axis_index("c")
    nch = idx_v.shape[0] // 6
    bpw = nch * CH
    rb_base = wid * (bpw // 8)

    pltpu.sync_copy(idx_hbm.at[wid], idx_v)

    def zrow(i, _):
        rb, r = i // 8, i % 8
        for p in range(2):
            for q in range(4):
                ebuf_v[p, rb, 1, r, pl.ds(64 + 16 * q, 16)] = (
                    jnp.zeros((16,), jnp.float32))
        return 0
    lax.fori_loop(0, (CH // 8) * 8, zrow, 0)

    def fire(k):
        copies = []
        for j in range(6):
            tab = pt_hbm if j < 2 else mt_hbm
            copies.append(
                pltpu.async_copy(tab.at[idx_v.at[j * nch + k]],
                                 gbuf_v.at[k % 2, j], gsem))
        return copies

    store = [None, None]
    gcop = [None, None]
    gcop[0] = fire(0)
    for k in range(nch):
        if k + 1 < nch:
            gcop[(k + 1) % 2] = fire(k + 1)
        for c in gcop[k % 2]:
            c.wait()
        if store[k % 2] is not None:
            store[k % 2].wait()

        ebuf = ebuf_v.at[k % 2]
        gbuf = gbuf_v.at[k % 2]

        @plsc.parallel_loop(0, CH, unroll=4)
        def rearrange(rr):
            rbq, rq = rr // 8, rr % 8
            for j in range(6):
                for h in range(2):
                    lq = 32 * j + 16 * h
                    v = gbuf[j, rr, pl.ds(16 * h, 16)]
                    ebuf[rbq, lq // 128, rq, pl.ds(lq % 128, 16)] = v

        store[k % 2] = pltpu.async_copy(
            ebuf_v.at[k % 2],
            out_hbm.at[pl.ds(rb_base + k * (CH // 8), CH // 8)], ssem)
    for cp in store:
        if cp is not None:
            cp.wait()


def _mlp_body(act_ref, x_ref, w1a_ref, w1b_ref, w1n_ref, b1_ref, w2_ref,
              b2_ref, w3_ref, b3_ref, out_ref):
    nb = act_ref.shape[0] * 8
    lo = act_ref[:, 0].reshape(nb, 128)
    hi = act_ref[:, 1].reshape(nb, 128)
    h1 = jnp.dot(lo, w1a_ref[:, :], preferred_element_type=jnp.float32)
    h1 += jnp.dot(hi, w1b_ref[:, :], preferred_element_type=jnp.float32)
    h1 += jnp.dot(x_ref[:, 6:10], w1n_ref[:, :],
                  preferred_element_type=jnp.float32)
    h1 = jnp.maximum(h1 + b1_ref[:, :], 0.0)
    h2 = jnp.maximum(jnp.dot(h1, w2_ref[:, :],
                             preferred_element_type=jnp.float32)
                     + b2_ref[:, :], 0.0)
    out_ref[:, :] = (jnp.dot(h2, w3_ref[:, :],
                             preferred_element_type=jnp.float32)
                     + b3_ref[:, :])


def kernel(x, pokemon_table, move_table, W1, b1, W2, b2, W3, b3):
    B = x.shape[0]
    bpw = B // NW
    nch = bpw // CH
    ids = x[:, :6].astype(jnp.int32)
    idx = ids.T.reshape(6, NW, nch, CH).transpose(1, 0, 2, 3)
    idx = idx.reshape(NW, 6 * nch, CH)

    acts = pl.kernel(
        _sc_gather_body,
        out_type=jax.ShapeDtypeStruct((B // 8, 2, 8, 128), jnp.float32),
        scratch_types=[
            pltpu.VMEM((6 * nch, CH), jnp.int32),
            pltpu.VMEM((2, 6, CH, 32), jnp.float32),
            pltpu.VMEM((2, CH // 8, 2, 8, 128), jnp.float32),
            pltpu.SemaphoreType.DMA,
            pltpu.SemaphoreType.DMA,
        ],
        mesh=plsc.VectorSubcoreMesh(core_axis_name="c", subcore_axis_name="s"),
        compiler_params=pltpu.CompilerParams(
            use_tc_tiling_on_sc=False, needs_layout_passes=False),
    )(pokemon_table, move_table, idx)

    W1a = W1[:128]
    W1b = jnp.pad(W1[128:192], ((0, 64), (0, 0)))
    W1n = W1[192:196]

    grid = B // BB
    full = lambda shape: pl.BlockSpec(shape, lambda i: (0,) * len(shape))
    return pl.pallas_call(
        _mlp_body,
        grid=(grid,),
        in_specs=[
            pl.BlockSpec((BB // 8, 2, 8, 128), lambda i: (i, 0, 0, 0)),
            pl.BlockSpec((BB, 10), lambda i: (i, 0)),
            full((128, 128)),
            full((128, 128)),
            full((4, 128)),
            full((1, 128)),
            full(W2.shape),
            full((1, 64)),
            full(W3.shape),
            full((1, 18)),
        ],
        out_specs=pl.BlockSpec((BB, 18), lambda i: (i, 0)),
        out_shape=jax.ShapeDtypeStruct((B, 18), jnp.float32),
    )(acts, x, W1a, W1b, W1n, b1.reshape(1, 128), W2, b2.reshape(1, 64), W3,
      b3.reshape(1, 18))
